# Initial kernel scaffold; baseline (speedup 1.0000x reference)
import jax
import jax.numpy as jnp
from jax.experimental import pallas as pl


def kernel(x, adj_t, W1, b1, W2, b2):
    raise NotImplementedError("write your pallas kernel here")


# speedup vs baseline: 15.2572x; 15.2572x over previous
import jax
import jax.numpy as jnp
from jax import lax
from jax.experimental import pallas as pl
from jax.experimental.pallas import tpu as pltpu
from jax.experimental.pallas import tpu_sc as plsc

_N = 10000
_E = 320000
_D = 128
_NT = 16
_NC = 2
_NW = _NT * _NC
_N_PAD = 10240
_RPT = _N_PAD // _NT
_EB = 128
_NB = _E // _EB
_ZR = 64

_mesh = plsc.VectorSubcoreMesh(core_axis_name="c", subcore_axis_name="s")


def _deg_body(dst_hbm, out_hbm, didx, ones_v, zbuf, acc):
    c = lax.axis_index("c")
    s = lax.axis_index("s")
    w = s * _NC + c

    one16 = jnp.ones((16,), jnp.float32)
    zero16 = jnp.zeros((16,), jnp.float32)

    def fill(i, _):
        ones_v[i, :] = one16
        return 0
    lax.fori_loop(0, _EB, fill, 0)

    def zfill(i, _):
        zbuf[i, :] = zero16
        return 0
    lax.fori_loop(0, _ZR, zfill, 0)

    base = s * _RPT

    def zcopy(i, _):
        pltpu.sync_copy(zbuf, acc.at[pl.ds(base + i * _ZR, _ZR), :])
        return 0
    lax.fori_loop(0, _RPT // _ZR, zcopy, 0)
    plsc.subcore_barrier()

    nb = jnp.where(w < _NB % _NW, _NB // _NW + 1, _NB // _NW)

    def body(i, _):
        e0 = (w + i * _NW) * _EB
        pltpu.sync_copy(dst_hbm.at[pl.ds(e0, _EB)], didx)
        pltpu.sync_copy(ones_v, acc.at[didx], add=True)
        return 0
    lax.fori_loop(0, nb, body, 0)
    plsc.subcore_barrier()

    pltpu.sync_copy(acc.at[pl.ds(base, _RPT), :],
                    out_hbm.at[pl.ds(c * _N_PAD + base, _RPT), :])


_deg_call = pl.kernel(
    _deg_body,
    out_type=jax.ShapeDtypeStruct((2 * _N_PAD, 16), jnp.float32),
    mesh=_mesh,
    scratch_types=[
        pltpu.VMEM((_EB,), jnp.int32),
        pltpu.VMEM((_EB, 16), jnp.float32),
        pltpu.VMEM((_ZR, 16), jnp.float32),
        pltpu.VMEM_SHARED((_N_PAD, 16), jnp.float32),
    ],
)


def _scat_body(g_hbm, src_hbm, dst_hbm, out_hbm, sidx, didx, rows, zbuf, acc,
               sem):
    c = lax.axis_index("c")
    s = lax.axis_index("s")
    w = s * _NC + c

    zero16 = jnp.zeros((16,), jnp.float32)

    def zfill(i, _):
        r = i // (_D // 16)
        col = (i % (_D // 16)) * 16
        zbuf[r, pl.ds(col, 16)] = zero16
        return 0
    lax.fori_loop(0, _ZR * (_D // 16), zfill, 0)

    base = s * _RPT

    def zcopy(i, _):
        pltpu.sync_copy(zbuf, acc.at[pl.ds(base + i * _ZR, _ZR), :])
        return 0
    lax.fori_loop(0, _RPT // _ZR, zcopy, 0)
    plsc.subcore_barrier()

    nb = jnp.where(w < _NB % _NW, _NB // _NW + 1, _NB // _NW)

    def body(i, _):
        e0 = (w + i * _NW) * _EB
        pltpu.sync_copy(src_hbm.at[pl.ds(e0, _EB)], sidx)
        pltpu.sync_copy(dst_hbm.at[pl.ds(e0, _EB)], didx)
        pltpu.async_copy(g_hbm.at[sidx], rows, sem).wait()
        pltpu.sync_copy(rows, acc.at[didx], add=True)
        return 0
    lax.fori_loop(0, nb, body, 0)
    plsc.subcore_barrier()

    pltpu.sync_copy(acc.at[pl.ds(base, _RPT), :],
                    out_hbm.at[pl.ds(c * _N_PAD + base, _RPT), :])


_scat_call = pl.kernel(
    _scat_body,
    out_type=jax.ShapeDtypeStruct((2 * _N_PAD, _D), jnp.float32),
    mesh=_mesh,
    scratch_types=[
        pltpu.VMEM((_EB,), jnp.int32),
        pltpu.VMEM((_EB,), jnp.int32),
        pltpu.VMEM((_EB, _D), jnp.float32),
        pltpu.VMEM((_ZR, _D), jnp.float32),
        pltpu.VMEM_SHARED((_N_PAD, _D), jnp.float32),
        pltpu.SemaphoreType.DMA,
    ],
)


_BN = 640
_GRID = _N_PAD // _BN


def _tc_a_body(x_ref, w_ref, d0_ref, d1_ref, g_ref, dinv_ref):
    deg = d0_ref[:, 0:1] + d1_ref[:, 0:1] + 1.0
    dinv = lax.rsqrt(deg)
    h = jnp.dot(x_ref[:, :], w_ref[:, :], preferred_element_type=jnp.float32)
    g_ref[:, :] = dinv * h
    dinv_ref[:, :] = jnp.broadcast_to(dinv, (_BN, _D))


_tc_a = pl.pallas_call(
    _tc_a_body,
    grid=(_GRID,),
    in_specs=[
        pl.BlockSpec((_BN, _D), lambda i: (i, 0)),
        pl.BlockSpec((_D, _D), lambda i: (0, 0)),
        pl.BlockSpec((_BN, 16), lambda i: (i, 0)),
        pl.BlockSpec((_BN, 16), lambda i: (i, 0)),
    ],
    out_specs=[pl.BlockSpec((_BN, _D), lambda i: (i, 0))] * 2,
    out_shape=[jax.ShapeDtypeStruct((_N_PAD, _D), jnp.float32)] * 2,
)


def _tc_b_body(s0_ref, s1_ref, g1_ref, dinv_ref, b1_ref, w2_ref, g2_ref):
    out1 = dinv_ref[:, :] * (s0_ref[:, :] + s1_ref[:, :] + g1_ref[:, :])
    out1 = out1 + b1_ref[:, :]
    h2 = jnp.dot(out1, w2_ref[:, :], preferred_element_type=jnp.float32)
    g2_ref[:, :] = dinv_ref[:, :] * h2


_tc_b = pl.pallas_call(
    _tc_b_body,
    grid=(_GRID,),
    in_specs=[
        pl.BlockSpec((_BN, _D), lambda i: (i, 0)),
        pl.BlockSpec((_BN, _D), lambda i: (i, 0)),
        pl.BlockSpec((_BN, _D), lambda i: (i, 0)),
        pl.BlockSpec((_BN, _D), lambda i: (i, 0)),
        pl.BlockSpec((1, _D), lambda i: (0, 0)),
        pl.BlockSpec((_D, _D), lambda i: (0, 0)),
    ],
    out_specs=pl.BlockSpec((_BN, _D), lambda i: (i, 0)),
    out_shape=jax.ShapeDtypeStruct((_N_PAD, _D), jnp.float32),
)


def _tc_c_body(s0_ref, s1_ref, g2_ref, dinv_ref, b2_ref, out_ref):
    out_ref[:, :] = (dinv_ref[:, :] * (s0_ref[:, :] + s1_ref[:, :] +
                                       g2_ref[:, :]) + b2_ref[:, :])


_tc_c = pl.pallas_call(
    _tc_c_body,
    grid=(_GRID,),
    in_specs=[
        pl.BlockSpec((_BN, _D), lambda i: (i, 0)),
        pl.BlockSpec((_BN, _D), lambda i: (i, 0)),
        pl.BlockSpec((_BN, _D), lambda i: (i, 0)),
        pl.BlockSpec((_BN, _D), lambda i: (i, 0)),
        pl.BlockSpec((1, _D), lambda i: (0, 0)),
    ],
    out_specs=pl.BlockSpec((_BN, _D), lambda i: (i, 0)),
    out_shape=jax.ShapeDtypeStruct((_N_PAD, _D), jnp.float32),
)


def kernel(x, adj_t, W1, b1, W2, b2):
    src = adj_t[0]
    dst = adj_t[1]
    xp = jnp.pad(x, ((0, _N_PAD - _N), (0, 0)))

    degp = _deg_call(dst)
    d0 = degp[:_N_PAD]
    d1 = degp[_N_PAD:]

    g1, dinvb = _tc_a(xp, W1, d0, d1)

    s1 = _scat_call(g1, src, dst)
    g2 = _tc_b(s1[:_N_PAD], s1[_N_PAD:], g1, dinvb, b1.reshape(1, _D), W2)

    s2 = _scat_call(g2, src, dst)
    out = _tc_c(s2[:_N_PAD], s2[_N_PAD:], g2, dinvb, b2.reshape(1, _D))
    return out[:_N]


# speedup vs baseline: 20.9998x; 1.3764x over previous
import jax
import jax.numpy as jnp
from jax import lax
from jax.experimental import pallas as pl
from jax.experimental.pallas import tpu as pltpu
from jax.experimental.pallas import tpu_sc as plsc

_N = 10000
_E = 320000
_D = 128
_NT = 16
_NC = 2
_NW = _NT * _NC
_N_PAD = 10240
_RPT = _N_PAD // _NT
_EB = 128
_NBW = 80
_NB_PAD = _NBW * _NW
_ZR = 16
_NBUF = 2

_PAD_DST = _N


def _deg_body(dst_hbm, out_hbm, didxb0, ones_v, zbuf, acc):
    c = lax.axis_index("c")
    s = lax.axis_index("s")
    w = s * _NC + c

    one16 = jnp.ones((16,), jnp.float32)
    zero16 = jnp.zeros((16,), jnp.float32)

    def fill(i, _):
        ones_v[i, :] = one16
        return 0
    lax.fori_loop(0, _EB, fill, 0)

    def zfill(i, _):
        zbuf[i, :] = zero16
        return 0
    lax.fori_loop(0, _ZR, zfill, 0)

    base = s * _RPT

    def zcopy(i, _):
        pltpu.sync_copy(zbuf, acc.at[pl.ds(base + i * _ZR, _ZR), :])
        return 0
    lax.fori_loop(0, _RPT // _ZR, zcopy, 0)
    plsc.subcore_barrier()

    ebase = w * _NBW * _EB

    def body(i, _):
        pltpu.sync_copy(dst_hbm.at[pl.ds(ebase + i * _EB, _EB)], didxb0)
        pltpu.sync_copy(ones_v, acc.at[didxb0], add=True)
        return 0
    lax.fori_loop(0, _NBW, body, 0)
    plsc.subcore_barrier()

    pltpu.sync_copy(acc.at[pl.ds(base, _RPT), :],
                    out_hbm.at[pl.ds(c * _N_PAD + base, _RPT), :])


def _scat_body(g_hbm, src_hbm, dst_hbm, out_hbm, sidx0, sidx1, didxb0, didxb1,
               rows0, rows1, zbuf, acc, gsem0, gsem1):
    c = lax.axis_index("c")
    s = lax.axis_index("s")
    w = s * _NC + c

    zero16 = jnp.zeros((16,), jnp.float32)

    def zfill(i, _):
        r = i // (_D // 16)
        col = (i % (_D // 16)) * 16
        zbuf[r, pl.ds(col, 16)] = zero16
        return 0
    lax.fori_loop(0, _ZR * (_D // 16), zfill, 0)

    base = s * _RPT

    def zcopy(i, _):
        pltpu.sync_copy(zbuf, acc.at[pl.ds(base + i * _ZR, _ZR), :])
        return 0
    lax.fori_loop(0, _RPT // _ZR, zcopy, 0)
    plsc.subcore_barrier()

    ebase = w * _NBW * _EB
    sidxb = (sidx0, sidx1)
    didxb = (didxb0, didxb1)
    rows = (rows0, rows1)
    gsems = (gsem0, gsem1)

    def load_and_fire(i, j):
        e0 = ebase + i * _EB
        pltpu.sync_copy(src_hbm.at[pl.ds(e0, _EB)], sidxb[j])
        pltpu.sync_copy(dst_hbm.at[pl.ds(e0, _EB)], didxb[j])
        pltpu.async_copy(g_hbm.at[sidxb[j]], rows[j], gsems[j])

    def emit(i, j, fire):
        pltpu.make_async_copy(g_hbm.at[sidxb[j]], rows[j], gsems[j]).wait()
        pltpu.sync_copy(rows[j], acc.at[didxb[j]], add=True)
        if fire:
            load_and_fire(i + 2, j)

    for j in range(2):
        load_and_fire(j, j)

    def round_body(r, _):
        for j in range(2):
            emit(r * 2 + j, j, True)
        return 0
    lax.fori_loop(0, _NBW // 2 - 1, round_body, 0)
    for j in range(2):
        emit(_NBW - 2 + j, j, False)
    plsc.subcore_barrier()

    pltpu.sync_copy(acc.at[pl.ds(base, _RPT), :],
                    out_hbm.at[pl.ds(c * _N_PAD + base, _RPT), :])


import functools


@functools.lru_cache(maxsize=None)
def _sc_kernels():
    mesh = plsc.VectorSubcoreMesh(core_axis_name="c", subcore_axis_name="s",
                                  num_cores=_NC, num_subcores=_NT)
    deg_call = pl.kernel(
        _deg_body,
        out_type=jax.ShapeDtypeStruct((2 * _N_PAD, 16), jnp.float32),
        mesh=mesh,
        scratch_types=[
            pltpu.VMEM((_EB,), jnp.int32),
            pltpu.VMEM((_EB, 16), jnp.float32),
            pltpu.VMEM((_ZR, 16), jnp.float32),
            pltpu.VMEM_SHARED((_N_PAD, 16), jnp.float32),
        ],
    )
    scat_call = pl.kernel(
        _scat_body,
        out_type=jax.ShapeDtypeStruct((2 * _N_PAD, _D), jnp.float32),
        mesh=mesh,
        scratch_types=(
            [pltpu.VMEM((_EB,), jnp.int32)] * 4 +
            [pltpu.VMEM((_EB, _D), jnp.float32)] * 2 +
            [pltpu.VMEM((_ZR, _D), jnp.float32),
             pltpu.VMEM_SHARED((_N_PAD, _D), jnp.float32)] +
            [pltpu.SemaphoreType.DMA] * 2
        ),
    )
    return deg_call, scat_call


_BN = 640
_GRID = _N_PAD // _BN


def _tc_a_body(x_ref, w_ref, d0_ref, d1_ref, g_ref, dinv_ref):
    deg = d0_ref[:, 0:1] + d1_ref[:, 0:1] + 1.0
    dinv = lax.rsqrt(deg)
    h = jnp.dot(x_ref[:, :], w_ref[:, :], preferred_element_type=jnp.float32)
    g_ref[:, :] = dinv * h
    dinv_ref[:, :] = jnp.broadcast_to(dinv, (_BN, _D))


_tc_a = pl.pallas_call(
    _tc_a_body,
    grid=(_GRID,),
    in_specs=[
        pl.BlockSpec((_BN, _D), lambda i: (i, 0)),
        pl.BlockSpec((_D, _D), lambda i: (0, 0)),
        pl.BlockSpec((_BN, 16), lambda i: (i, 0)),
        pl.BlockSpec((_BN, 16), lambda i: (i, 0)),
    ],
    out_specs=[pl.BlockSpec((_BN, _D), lambda i: (i, 0))] * 2,
    out_shape=[jax.ShapeDtypeStruct((_N_PAD, _D), jnp.float32)] * 2,
)


def _tc_bc_body(s0_ref, s1_ref, g_ref, dinv_ref, b_ref, wn_ref, fl_ref,
                out_ref):
    comb = dinv_ref[:, :] * (s0_ref[:, :] + s1_ref[:, :] + g_ref[:, :])
    comb = comb + b_ref[:, :]
    h = jnp.dot(comb, wn_ref[:, :], preferred_element_type=jnp.float32)
    mult = fl_ref[:, :] + (1.0 - fl_ref[:, :]) * dinv_ref[:, :]
    out_ref[:, :] = mult * h


_tc_bc = pl.pallas_call(
    _tc_bc_body,
    grid=(_GRID,),
    in_specs=[
        pl.BlockSpec((_BN, _D), lambda i: (i, 0)),
        pl.BlockSpec((_BN, _D), lambda i: (i, 0)),
        pl.BlockSpec((_BN, _D), lambda i: (i, 0)),
        pl.BlockSpec((_BN, _D), lambda i: (i, 0)),
        pl.BlockSpec((1, _D), lambda i: (0, 0)),
        pl.BlockSpec((_D, _D), lambda i: (0, 0)),
        pl.BlockSpec((1, _D), lambda i: (0, 0)),
    ],
    out_specs=pl.BlockSpec((_BN, _D), lambda i: (i, 0)),
    out_shape=jax.ShapeDtypeStruct((_N_PAD, _D), jnp.float32),
)


def kernel(x, adj_t, W1, b1, W2, b2):
    src = adj_t[0]
    dst = adj_t[1]
    pad_e = _NB_PAD * _EB - _E
    cyc = jnp.arange(pad_e, dtype=dst.dtype)
    src1 = jnp.concatenate([src, cyc % _N])
    dst1 = jnp.concatenate([dst, _PAD_DST + cyc % (_N_PAD - _N)])
    xp = jnp.pad(x, ((0, _N_PAD - _N), (0, 0)))

    _deg_call, _scat_call = _sc_kernels()
    degp = _deg_call(dst1)
    d0 = degp[:_N_PAD]
    d1 = degp[_N_PAD:]

    g1, dinvb = _tc_a(xp, W1, d0, d1)

    wn = jnp.stack([W2, jnp.eye(_D, dtype=jnp.float32)])
    bs = jnp.stack([b1.reshape(1, _D), b2.reshape(1, _D)])
    fls = jnp.stack([jnp.zeros((1, _D), jnp.float32),
                     jnp.ones((1, _D), jnp.float32)])

    def layer(g, xs):
        wn_i, b_i, fl_i = xs
        s = _scat_call(g, src1, dst1)
        g_next = _tc_bc(s[:_N_PAD], s[_N_PAD:], g, dinvb, b_i, wn_i, fl_i)
        return g_next, None

    out, _ = lax.scan(layer, g1, (wn, bs, fls))
    return out[:_N]


# speedup vs baseline: 22.8010x; 1.0858x over previous
import jax
import jax.numpy as jnp
from jax import lax
from jax.experimental import pallas as pl
from jax.experimental.pallas import tpu as pltpu
from jax.experimental.pallas import tpu_sc as plsc

_N = 10000
_E = 320000
_D = 128
_NT = 16
_NC = 2
_NW = _NT * _NC
_N_PAD = 10240
_RPT = _N_PAD // _NT
_EB = 128
_NBW = 80
_NB_PAD = _NBW * _NW
_ZR = 16
_NBUF = 2

_PAD_DST = _N


def _deg_body(dst_hbm, out_hbm, didxb0, ones_v, zbuf, acc):
    c = lax.axis_index("c")
    s = lax.axis_index("s")
    w = s * _NC + c

    one16 = jnp.ones((16,), jnp.float32)
    zero16 = jnp.zeros((16,), jnp.float32)

    def fill(i, _):
        ones_v[i, :] = one16
        return 0
    lax.fori_loop(0, _EB, fill, 0)

    def zfill(i, _):
        zbuf[i, :] = zero16
        return 0
    lax.fori_loop(0, _ZR, zfill, 0)

    base = s * _RPT

    def zcopy(i, _):
        pltpu.sync_copy(zbuf, acc.at[pl.ds(base + i * _ZR, _ZR), :])
        return 0
    lax.fori_loop(0, _RPT // _ZR, zcopy, 0)
    plsc.subcore_barrier()

    ebase = w * _NBW * _EB

    def body(i, _):
        pltpu.sync_copy(dst_hbm.at[pl.ds(ebase + i * _EB, _EB)], didxb0)
        pltpu.sync_copy(ones_v, acc.at[didxb0], add=True)
        return 0
    lax.fori_loop(0, _NBW, body, 0)
    plsc.subcore_barrier()

    pltpu.sync_copy(acc.at[pl.ds(base, _RPT), :],
                    out_hbm.at[pl.ds(c * _N_PAD + base, _RPT), :])


_CH = 8
_NCH = _NBW // _CH


def _scat_body(g_hbm, src_hbm, dst_hbm, out_hbm, sslab, dslab,
               rows0, rows1, zbuf, acc, gsem0, gsem1):
    c = lax.axis_index("c")
    s = lax.axis_index("s")
    w = s * _NC + c

    zero16 = jnp.zeros((16,), jnp.float32)

    def zfill(i, _):
        r = i // (_D // 16)
        col = (i % (_D // 16)) * 16
        zbuf[r, pl.ds(col, 16)] = zero16
        return 0
    lax.fori_loop(0, _ZR * (_D // 16), zfill, 0)

    base = s * _RPT

    def zcopy(i, _):
        pltpu.sync_copy(zbuf, acc.at[pl.ds(base + i * _ZR, _ZR), :])
        return 0
    lax.fori_loop(0, _RPT // _ZR, zcopy, 0)
    plsc.subcore_barrier()

    rbase = w * _NBW
    rows = (rows0, rows1)
    gsems = (gsem0, gsem1)

    def chunk_body(r, _):
        r0 = rbase + r * _CH
        pltpu.sync_copy(src_hbm.at[pl.ds(r0, _CH), :], sslab)
        pltpu.sync_copy(dst_hbm.at[pl.ds(r0, _CH), :], dslab)
        for k in range(2):
            pltpu.async_copy(g_hbm.at[sslab.at[k]], rows[k], gsems[k])
        for j in range(_CH):
            k = j % 2
            pltpu.make_async_copy(g_hbm.at[sslab.at[j]], rows[k],
                                  gsems[k]).wait()
            pltpu.sync_copy(rows[k], acc.at[dslab.at[j]], add=True)
            if j + 2 < _CH:
                pltpu.async_copy(g_hbm.at[sslab.at[j + 2]], rows[k], gsems[k])
        return 0
    lax.fori_loop(0, _NCH, chunk_body, 0)
    plsc.subcore_barrier()

    pltpu.sync_copy(acc.at[pl.ds(base, _RPT), :],
                    out_hbm.at[pl.ds(c * _N_PAD + base, _RPT), :])


import functools


@functools.lru_cache(maxsize=None)
def _sc_kernels():
    mesh = plsc.VectorSubcoreMesh(core_axis_name="c", subcore_axis_name="s",
                                  num_cores=_NC, num_subcores=_NT)
    deg_call = pl.kernel(
        _deg_body,
        out_type=jax.ShapeDtypeStruct((2 * _N_PAD, 16), jnp.float32),
        mesh=mesh,
        scratch_types=[
            pltpu.VMEM((_EB,), jnp.int32),
            pltpu.VMEM((_EB, 16), jnp.float32),
            pltpu.VMEM((_ZR, 16), jnp.float32),
            pltpu.VMEM_SHARED((_N_PAD, 16), jnp.float32),
        ],
    )
    scat_call = pl.kernel(
        _scat_body,
        out_type=jax.ShapeDtypeStruct((2 * _N_PAD, _D), jnp.float32),
        mesh=mesh,
        scratch_types=(
            [pltpu.VMEM((_CH, _EB), jnp.int32)] * 2 +
            [pltpu.VMEM((_EB, _D), jnp.float32)] * 2 +
            [pltpu.VMEM((_ZR, _D), jnp.float32),
             pltpu.VMEM_SHARED((_N_PAD, _D), jnp.float32)] +
            [pltpu.SemaphoreType.DMA] * 2
        ),
    )
    return deg_call, scat_call


_BN = 640
_GRID = _N_PAD // _BN


def _tc_a_body(x_ref, w_ref, d0_ref, d1_ref, g_ref, dinv_ref):
    deg = d0_ref[:, 0:1] + d1_ref[:, 0:1] + 1.0
    dinv = lax.rsqrt(deg)
    h = jnp.dot(x_ref[:, :], w_ref[:, :], preferred_element_type=jnp.float32)
    g_ref[:, :] = dinv * h
    dinv_ref[:, :] = jnp.broadcast_to(dinv, (_BN, _D))


_tc_a = pl.pallas_call(
    _tc_a_body,
    grid=(_GRID,),
    in_specs=[
        pl.BlockSpec((_BN, _D), lambda i: (i, 0)),
        pl.BlockSpec((_D, _D), lambda i: (0, 0)),
        pl.BlockSpec((_BN, 16), lambda i: (i, 0)),
        pl.BlockSpec((_BN, 16), lambda i: (i, 0)),
    ],
    out_specs=[pl.BlockSpec((_BN, _D), lambda i: (i, 0))] * 2,
    out_shape=[jax.ShapeDtypeStruct((_N_PAD, _D), jnp.float32)] * 2,
)


def _tc_bc_body(s0_ref, s1_ref, g_ref, dinv_ref, b_ref, wn_ref, fl_ref,
                out_ref):
    comb = dinv_ref[:, :] * (s0_ref[:, :] + s1_ref[:, :] + g_ref[:, :])
    comb = comb + b_ref[:, :]
    h = jnp.dot(comb, wn_ref[:, :], preferred_element_type=jnp.float32)
    mult = fl_ref[:, :] + (1.0 - fl_ref[:, :]) * dinv_ref[:, :]
    out_ref[:, :] = mult * h


_tc_bc = pl.pallas_call(
    _tc_bc_body,
    grid=(_GRID,),
    in_specs=[
        pl.BlockSpec((_BN, _D), lambda i: (i, 0)),
        pl.BlockSpec((_BN, _D), lambda i: (i, 0)),
        pl.BlockSpec((_BN, _D), lambda i: (i, 0)),
        pl.BlockSpec((_BN, _D), lambda i: (i, 0)),
        pl.BlockSpec((1, _D), lambda i: (0, 0)),
        pl.BlockSpec((_D, _D), lambda i: (0, 0)),
        pl.BlockSpec((1, _D), lambda i: (0, 0)),
    ],
    out_specs=pl.BlockSpec((_BN, _D), lambda i: (i, 0)),
    out_shape=jax.ShapeDtypeStruct((_N_PAD, _D), jnp.float32),
)


def kernel(x, adj_t, W1, b1, W2, b2):
    src = adj_t[0]
    dst = adj_t[1]
    pad_e = _NB_PAD * _EB - _E
    cyc = jnp.arange(pad_e, dtype=dst.dtype)
    src1 = jnp.concatenate([src, cyc % _N])
    dst1 = jnp.concatenate([dst, _PAD_DST + cyc % (_N_PAD - _N)])
    xp = jnp.pad(x, ((0, _N_PAD - _N), (0, 0)))

    _deg_call, _scat_call = _sc_kernels()
    degp = _deg_call(dst1)
    d0 = degp[:_N_PAD]
    d1 = degp[_N_PAD:]

    g1, dinvb = _tc_a(xp, W1, d0, d1)

    wn = jnp.stack([W2, jnp.eye(_D, dtype=jnp.float32)])
    bs = jnp.stack([b1.reshape(1, _D), b2.reshape(1, _D)])
    fls = jnp.stack([jnp.zeros((1, _D), jnp.float32),
                     jnp.ones((1, _D), jnp.float32)])

    src2 = src1.reshape(_NB_PAD, _EB)
    dst2 = dst1.reshape(_NB_PAD, _EB)

    def layer(g, xs):
        wn_i, b_i, fl_i = xs
        s = _scat_call(g, src2, dst2)
        g_next = _tc_bc(s[:_N_PAD], s[_N_PAD:], g, dinvb, b_i, wn_i, fl_i)
        return g_next, None

    out, _ = lax.scan(layer, g1, (wn, bs, fls))
    return out[:_N]
